# Initial kernel scaffold; baseline (speedup 1.0000x reference)
import jax
import jax.numpy as jnp
from jax.experimental import pallas as pl


def kernel(categories, table, W, b):
    raise NotImplementedError("write your pallas kernel here")


# speedup vs baseline: 1.7238x; 1.7238x over previous
import functools

import jax
import jax.numpy as jnp
from jax import lax
from jax.experimental import pallas as pl
from jax.experimental.pallas import tpu as pltpu
from jax.experimental.pallas import tpu_sc as plsc


def _proj_body(table_ref, w_ref, b_ref, out_ref):
    out_ref[...] = (
        jnp.dot(table_ref[...], w_ref[...], preferred_element_type=jnp.float32)
        + b_ref[...]
    )


def _project_table(table, W, b):
    V, _ = table.shape
    N = W.shape[1]
    return pl.pallas_call(
        _proj_body,
        out_shape=jax.ShapeDtypeStruct((V, N), jnp.float32),
    )(table, W, b.reshape(1, N))


def _make_sc_gather(n_rows, D, V, n_workers, chunk):
    per_w = n_rows // n_workers
    n_chunks = per_w // chunk
    mesh = plsc.VectorSubcoreMesh(core_axis_name="c", subcore_axis_name="s")
    num_cores = 2

    @functools.partial(
        pl.kernel,
        mesh=mesh,
        out_type=jax.ShapeDtypeStruct((n_rows, D), jnp.float32),
        scratch_types=[
            pltpu.VMEM((n_chunks, chunk), jnp.int32),
            pltpu.VMEM((2, chunk, D), jnp.float32),
            pltpu.SemaphoreType.DMA,
            pltpu.SemaphoreType.DMA,
            pltpu.SemaphoreType.DMA,
        ],
    )
    def sc_gather(idx_hbm, proj_hbm, out_hbm, idx_v, rows_v, gsem, osem0, osem1):
        wid = lax.axis_index("s") * num_cores + lax.axis_index("c")
        base = wid * per_w
        pltpu.sync_copy(idx_hbm.at[wid], idx_v)

        osems = (osem0, osem1)

        def gather_chunk(c, buf):
            pltpu.async_copy(proj_hbm.at[idx_v.at[c]], rows_v.at[buf], gsem).wait()

        def start_out(c, buf):
            pltpu.async_copy(
                rows_v.at[buf],
                out_hbm.at[pl.ds(base + c * chunk, chunk)],
                osems[buf],
            )

        def wait_out(c_prev, buf):
            pltpu.make_async_copy(
                rows_v.at[buf],
                out_hbm.at[pl.ds(base + c_prev * chunk, chunk)],
                osems[buf],
            ).wait()

        gather_chunk(0, 0)
        start_out(0, 0)
        gather_chunk(1, 1)
        start_out(1, 1)

        def body(c0):
            for off in range(2):
                c = c0 + off
                buf = off
                wait_out(c - 2, buf)
                gather_chunk(c, buf)
                start_out(c, buf)

        pl.loop(2, n_chunks, step=2)(body)

        wait_out(n_chunks - 2, 0)
        wait_out(n_chunks - 1, 1)

    return sc_gather


def kernel(categories, table, W, b):
    B, L = categories.shape
    V, _ = table.shape
    D = W.shape[1]
    n_rows = B * L

    n_workers = 32
    chunk = 128
    assert n_rows % (n_workers * chunk) == 0

    proj = _project_table(table, W, b)
    idx = categories.reshape(n_workers, n_rows // (n_workers * chunk), chunk)
    idx = idx.astype(jnp.int32)
    out_flat = _make_sc_gather(n_rows, D, V, n_workers, chunk)(idx, proj)
    return out_flat.reshape(B, L, D)


# speedup vs baseline: 4.7879x; 2.7775x over previous
import functools

import jax
import jax.numpy as jnp
from jax import lax
from jax.experimental import pallas as pl
from jax.experimental.pallas import tpu as pltpu
from jax.experimental.pallas import tpu_sc as plsc


def _proj_body(table_ref, w_ref, b_ref, out_ref):
    out_ref[...] = (
        jnp.dot(table_ref[...], w_ref[...], preferred_element_type=jnp.float32)
        + b_ref[...]
    )


def _project_table(table, W, b):
    V, _ = table.shape
    N = W.shape[1]
    return pl.pallas_call(
        _proj_body,
        out_shape=jax.ShapeDtypeStruct((V, N), jnp.float32),
    )(table, W, b.reshape(1, N))


def _make_sc_gather(n_rows, D, V, n_workers, chunk):
    per_w = n_rows // n_workers
    n_chunks = per_w // chunk
    mesh = plsc.VectorSubcoreMesh(core_axis_name="c", subcore_axis_name="s")
    num_cores = 2

    @functools.partial(
        pl.kernel,
        mesh=mesh,
        out_type=jax.ShapeDtypeStruct((n_rows, D), jnp.float32),
        scratch_types=[
            pltpu.VMEM((n_chunks, chunk), jnp.int32),
            pltpu.VMEM((2, chunk, D), jnp.float32),
            pltpu.VMEM_SHARED((V, D), jnp.float32),
            pltpu.SemaphoreType.DMA,
            pltpu.SemaphoreType.DMA,
            pltpu.SemaphoreType.DMA,
        ],
    )
    def sc_gather(idx_hbm, proj_hbm, out_hbm, idx_v, rows_v, tab_sh, gsem, osem0, osem1):
        wid = lax.axis_index("s") * num_cores + lax.axis_index("c")
        base = wid * per_w
        @pl.when(lax.axis_index("s") == 0)
        def _stage_table():
            pltpu.sync_copy(proj_hbm, tab_sh)

        pltpu.sync_copy(idx_hbm.at[wid], idx_v)
        plsc.subcore_barrier()

        osems = (osem0, osem1)

        def gather_chunk(c, buf):
            pltpu.async_copy(tab_sh.at[idx_v.at[c]], rows_v.at[buf], gsem).wait()

        def start_out(c, buf):
            pltpu.async_copy(
                rows_v.at[buf],
                out_hbm.at[pl.ds(base + c * chunk, chunk)],
                osems[buf],
            )

        def wait_out(c_prev, buf):
            pltpu.make_async_copy(
                rows_v.at[buf],
                out_hbm.at[pl.ds(base + c_prev * chunk, chunk)],
                osems[buf],
            ).wait()

        gather_chunk(0, 0)
        start_out(0, 0)
        gather_chunk(1, 1)
        start_out(1, 1)

        def body(c0):
            for off in range(2):
                c = c0 + off
                buf = off
                wait_out(c - 2, buf)
                gather_chunk(c, buf)
                start_out(c, buf)

        pl.loop(2, n_chunks, step=2)(body)

        wait_out(n_chunks - 2, 0)
        wait_out(n_chunks - 1, 1)

    return sc_gather


def kernel(categories, table, W, b):
    B, L = categories.shape
    V, _ = table.shape
    D = W.shape[1]
    n_rows = B * L

    n_workers = 32
    chunk = 128
    assert n_rows % (n_workers * chunk) == 0

    proj = _project_table(table, W, b)
    idx = categories.reshape(n_workers, n_rows // (n_workers * chunk), chunk)
    idx = idx.astype(jnp.int32)
    out_flat = _make_sc_gather(n_rows, D, V, n_workers, chunk)(idx, proj)
    return out_flat.reshape(B, L, D)


# speedup vs baseline: 8.5664x; 1.7892x over previous
import functools

import jax
import jax.numpy as jnp
from jax import lax
from jax.experimental import pallas as pl
from jax.experimental.pallas import tpu as pltpu
from jax.experimental.pallas import tpu_sc as plsc


def _proj_body(table_ref, w_ref, b_ref, out_ref):
    out_ref[...] = (
        jnp.dot(table_ref[...], w_ref[...], preferred_element_type=jnp.float32)
        + b_ref[...]
    )


def _project_table(table, W, b):
    V, _ = table.shape
    N = W.shape[1]
    return pl.pallas_call(
        _proj_body,
        out_shape=jax.ShapeDtypeStruct((V, N), jnp.float32),
    )(table, W, b.reshape(1, N))


def _make_sc_gather(B, L, D, V, n_workers, chunk_b, idx_row):
    b_per_w = B // n_workers
    n_chunks = b_per_w // chunk_b
    rows_per_chunk = chunk_b * L
    mesh = plsc.VectorSubcoreMesh(core_axis_name="c", subcore_axis_name="s")
    num_cores = 2

    @functools.partial(
        pl.kernel,
        mesh=mesh,
        out_type=jax.ShapeDtypeStruct((B, L, D), jnp.float32),
        compiler_params=pltpu.CompilerParams(use_tc_tiling_on_sc=True),
        scratch_types=[
            pltpu.VMEM((n_chunks, idx_row), jnp.int32),
            pltpu.VMEM((2, idx_row, D), jnp.float32),
            pltpu.VMEM_SHARED((V, D), jnp.float32),
            pltpu.SemaphoreType.DMA,
            pltpu.SemaphoreType.DMA,
            pltpu.SemaphoreType.DMA,
        ],
    )
    def sc_gather(idx_hbm, proj_hbm, out_hbm, idx_v, rows_v, tab_sh, gsem, osem0, osem1):
        wid = lax.axis_index("s") * num_cores + lax.axis_index("c")
        base_b = wid * b_per_w

        @pl.when(lax.axis_index("s") == 0)
        def _stage_table():
            pltpu.sync_copy(proj_hbm, tab_sh)

        pltpu.sync_copy(idx_hbm.at[wid], idx_v)
        plsc.subcore_barrier()

        osems = (osem0, osem1)

        def gather_chunk(c, buf):
            pltpu.async_copy(tab_sh.at[idx_v.at[c]], rows_v.at[buf], gsem).wait()

        def out_copies(c, buf):
            b0 = base_b + c * chunk_b
            return [
                pltpu.make_async_copy(
                    rows_v.at[buf, pl.ds(i * L, L)],
                    out_hbm.at[b0 + i],
                    osems[buf],
                )
                for i in range(chunk_b)
            ]

        def start_out(c, buf):
            for cp in out_copies(c, buf):
                cp.start()

        def wait_out(c_prev, buf):
            for cp in out_copies(c_prev, buf):
                cp.wait()

        gather_chunk(0, 0)
        start_out(0, 0)
        gather_chunk(1, 1)
        start_out(1, 1)

        def body(c0):
            for off in range(2):
                c = c0 + off
                buf = off
                wait_out(c - 2, buf)
                gather_chunk(c, buf)
                start_out(c, buf)

        pl.loop(2, n_chunks, step=2)(body)

        wait_out(n_chunks - 2, 0)
        wait_out(n_chunks - 1, 1)

    return sc_gather


def kernel(categories, table, W, b):
    B, L = categories.shape
    V, _ = table.shape
    D = W.shape[1]

    n_workers = 32
    chunk_b = 2
    idx_row = 128
    assert B % (n_workers * chunk_b) == 0 and chunk_b * L <= idx_row

    proj = _project_table(table, W, b)
    idx = categories.astype(jnp.int32).reshape(B // chunk_b, chunk_b * L)
    idx = jnp.pad(idx, ((0, 0), (0, idx_row - chunk_b * L)))
    idx = idx.reshape(n_workers, B // (n_workers * chunk_b), idx_row)
    return _make_sc_gather(B, L, D, V, n_workers, chunk_b, idx_row)(idx, proj)
